# baseline (speedup 1.0000x reference)
import functools

import jax
import jax.numpy as jnp
from jax import lax
from jax.experimental import pallas as pl
from jax.experimental.pallas import tpu as pltpu
from jax.experimental.pallas import tpu_sc as plsc

_NUM_CORES = 2
_NUM_SUBCORES = 16
_NUM_WORKERS = _NUM_CORES * _NUM_SUBCORES
_LANES = 16


def _round_up(x: int, m: int) -> int:
    return (x + m - 1) // m * m


def _mask_body(chunk: int, ids_hbm, mask_hbm, ids_v, chunk_v):
    cid = lax.axis_index("c")
    sid = lax.axis_index("s")
    wid = sid * _NUM_CORES + cid
    base = wid * chunk

    pltpu.sync_copy(ids_hbm, ids_v)

    neg_inf = jnp.full((_LANES,), -jnp.inf, dtype=jnp.float32)

    def fill(i, carry):
        chunk_v[pl.ds(i * _LANES, _LANES)] = neg_inf
        return carry

    lax.fori_loop(0, chunk // _LANES, fill, 0)

    zeros = jnp.zeros((_LANES,), dtype=jnp.float32)
    n_ids = ids_v.shape[0]

    def scatter(i, carry):
        idx = ids_v[pl.ds(i * _LANES, _LANES)]
        loc = idx - base
        in_range = (loc >= 0) & (loc < chunk)
        loc = jnp.clip(loc, 0, chunk - 1)
        plsc.store_scatter(chunk_v, [loc], zeros, mask=in_range)
        return carry

    lax.fori_loop(0, n_ids // _LANES, scatter, 0)

    pltpu.sync_copy(chunk_v, mask_hbm.at[pl.ds(base, chunk)])


def _build_mask(allowed_token_ids, vocab_pad: int, chunk: int):
    n_ids = allowed_token_ids.shape[0]
    mesh = plsc.VectorSubcoreMesh(core_axis_name="c", subcore_axis_name="s")
    return pl.kernel(
        functools.partial(_mask_body, chunk),
        out_type=jax.ShapeDtypeStruct((vocab_pad,), jnp.float32),
        mesh=mesh,
        scratch_types=[
            pltpu.VMEM((n_ids,), jnp.int32),
            pltpu.VMEM((chunk,), jnp.float32),
        ],
        compiler_params=pltpu.CompilerParams(needs_layout_passes=False),
        name="sc_build_vocab_mask",
    )(allowed_token_ids)


def _add_body(mask_ref, scores_ref, out_ref):
    out_ref[...] = scores_ref[...] + mask_ref[...]


def kernel(input_ids, scores, allowed_token_ids):
    del input_ids
    batch, vocab = scores.shape
    ids = allowed_token_ids.astype(jnp.int32)

    chunk = _round_up(-(-vocab // _NUM_WORKERS), 16)
    vocab_pad = chunk * _NUM_WORKERS

    mask = _build_mask(ids, vocab_pad, chunk)
    mask2d = mask.reshape(1, vocab_pad)

    blk_v = 4096
    grid = -(-vocab_pad // blk_v)
    out = pl.pallas_call(
        _add_body,
        grid=(grid,),
        in_specs=[
            pl.BlockSpec((1, blk_v), lambda i: (0, i)),
            pl.BlockSpec((batch, blk_v), lambda i: (0, i)),
        ],
        out_specs=pl.BlockSpec((batch, blk_v), lambda i: (0, i)),
        out_shape=jax.ShapeDtypeStruct((batch, vocab), jnp.float32),
        name="tc_mask_add",
    )(mask2d, scores)
    return out
